# baseline (speedup 1.0000x reference)
import functools

import jax
import jax.numpy as jnp
from jax import lax
from jax.experimental import pallas as pl
from jax.experimental.pallas import tpu as pltpu
from jax.experimental.pallas import tpu_sc as plsc

D_MODEL = 1_000_000
HIDDEN = 64
OUT = 64

_B = 4096
_L = 200
_LT = _L // 8
_BT = _B // 128


_BN = 16384


def _transform_body(tT_ref, w_ref, b_ref, o_ref):
    h = jnp.maximum(tT_ref[...], 0.0)
    res = lax.dot_general(h, w_ref[...], (((0,), (1,)), ((), ())),
                          preferred_element_type=jnp.float32)
    o_ref[:, 0:OUT] = res + b_ref[...]


def _transform_table(tableT, W, b):
    n_rows = tableT.shape[1]
    grid = (n_rows + _BN - 1) // _BN
    return pl.pallas_call(
        _transform_body,
        grid=(grid,),
        in_specs=[
            pl.BlockSpec((HIDDEN, _BN), lambda i: (0, i)),
            pl.BlockSpec((OUT, HIDDEN), lambda i: (0, 0)),
            pl.BlockSpec((1, OUT), lambda i: (0, 0)),
        ],
        out_specs=pl.BlockSpec((_BN, 2 * OUT), lambda i: (i, 0)),
        out_shape=jax.ShapeDtypeStruct((n_rows, 2 * OUT), jnp.float32),
    )(tableT, W, b.reshape(1, OUT))


def _make_gather(n_lt):
    n_l = n_lt * 8
    mesh = plsc.VectorSubcoreMesh(core_axis_name="c", subcore_axis_name="s")

    @functools.partial(
        pl.kernel,
        mesh=mesh,
        out_type=jax.ShapeDtypeStruct((n_l * _B // 8, 8, 128), jnp.float32),
        scratch_types=[
            pltpu.VMEM((n_lt, 8, 128), jnp.int32),
            pltpu.VMEM((128, OUT), jnp.float32),
            pltpu.VMEM((128, OUT), jnp.float32),
            pltpu.VMEM((16, 8, OUT), jnp.float32),
            pltpu.VMEM((16, 8, OUT), jnp.float32),
            pltpu.SemaphoreType.DMA,
            pltpu.SemaphoreType.DMA,
            pltpu.SemaphoreType.DMA,
            pltpu.SemaphoreType.DMA,
        ],
        compiler_params=pltpu.CompilerParams(use_tc_tiling_on_sc=False),
    )
    def gather(table2_hbm, idx_hbm, out_hbm, idx_v, rows_a, rows_b, cb_a, cb_b,
               g0, g1, w0, w1):
        wid = lax.axis_index("s") * 2 + lax.axis_index("c")

        for lt in range(n_lt):
            pltpu.sync_copy(idx_hbm.at[lt, wid], idx_v.at[lt])

        def dbl(lt, c):
            for lr in range(8):
                for v in range(8):
                    sl = pl.ds(16 * v, 16)
                    idx_v[lt, lr, sl] = idx_v[lt, lr, sl] * 2
            return c

        lax.fori_loop(0, n_lt, dbl, 0, unroll=False)

        gsem = (g0, g1)
        wsem = (w0, w1)
        rbuf = (rows_a, rows_b)
        cbuf = (cb_a, cb_b)
        base = wid * 128

        def fire_gather(l, buf):
            pltpu.async_copy(table2_hbm.at[idx_v.at[l // 8, l % 8]],
                             rbuf[buf], gsem[buf])

        def wait_gather(l, buf):
            pltpu.make_async_copy(table2_hbm.at[idx_v.at[l // 8, l % 8]],
                                  rbuf[buf], gsem[buf]).wait()

        def fire_write(l, buf):
            pltpu.async_copy(cbuf[buf],
                             out_hbm.at[pl.ds((l * _B + base) // 8, 16), :,
                                        pl.ds(0, OUT)],
                             wsem[buf])

        def wait_write(l, buf):
            pltpu.make_async_copy(cbuf[buf],
                                  out_hbm.at[pl.ds((l * _B + base) // 8, 16), :,
                                             pl.ds(0, OUT)],
                                  wsem[buf]).wait()

        def compact(buf):
            rows = rbuf[buf]
            cb = cbuf[buf]
            for r in range(128):
                for q in range(4):
                    cb[r // 8, r % 8, pl.ds(q * 16, 16)] = rows[r, pl.ds(q * 16, 16)]

        def step(l, buf, last):
            wait_gather(l, buf)
            if not last:
                wait_write(l - 1, 1 - buf)
                fire_gather(l + 1, 1 - buf)
            compact(buf)
            fire_write(l, buf)

        fire_gather(jnp.int32(0), 0)
        wait_gather(jnp.int32(0), 0)
        fire_gather(jnp.int32(1), 1)
        compact(0)
        fire_write(jnp.int32(0), 0)

        def body(i, c):
            step(2 * i + 1, 1, False)
            step(2 * i + 2, 0, False)
            return c

        lax.fori_loop(0, (n_l - 2) // 2, body, 0, unroll=False)
        step(jnp.int32(n_l - 1), 1, True)
        wait_write(jnp.int32(n_l - 2), 0)
        wait_write(jnp.int32(n_l - 1), 1)

    return gather


_LBLK = 8


def _emit_body(e_ref, i_ref, o_ref):
    for k in range(_LBLK):
        a = e_ref[k].reshape(_B, 128)[:, 0:OUT]
        o_ref[k] = lax.dot_general(i_ref[...], a,
                                   (((1,), (1,)), ((), ())),
                                   preferred_element_type=jnp.float32)


def _emit(embL, ident, n_l):
    emb4 = embL.reshape(n_l, _B // 8, 8, 128)
    return pl.pallas_call(
        _emit_body,
        grid=(n_l // _LBLK,),
        in_specs=[
            pl.BlockSpec((_LBLK, _B // 8, 8, 128), lambda l: (l, 0, 0, 0)),
            pl.BlockSpec((OUT, OUT), lambda l: (0, 0)),
        ],
        out_specs=pl.BlockSpec((_LBLK, OUT, _B), lambda l: (l, 0, 0)),
        out_shape=jax.ShapeDtypeStruct((n_l, OUT, _B), jnp.float32),
    )(emb4, ident)


def kernel(inputs, table, W, b):
    tableT = table.T
    table2 = _transform_table(tableT, W, b)

    idx5 = (inputs.astype(jnp.int32).T
            .reshape(_LT, 8, _BT, 128)
            .transpose(0, 2, 1, 3))

    table2h = table2.reshape(2 * D_MODEL, OUT)
    ident = jnp.eye(OUT, dtype=jnp.float32)

    pieces = []
    for lo, hi in ((0, 13), (13, 25)):
        idx_c = lax.slice_in_dim(idx5, lo, hi, axis=0)
        emb_c = _make_gather(hi - lo)(table2h, idx_c)
        pieces.append(_emit(emb_c, ident, (hi - lo) * 8))
    out3 = jnp.concatenate(pieces, axis=0)

    return out3.transpose(2, 0, 1)


# speedup vs baseline: 1.1624x; 1.1624x over previous
import functools

import jax
import jax.numpy as jnp
from jax import lax
from jax.experimental import pallas as pl
from jax.experimental.pallas import tpu as pltpu
from jax.experimental.pallas import tpu_sc as plsc

D_MODEL = 1_000_000
HIDDEN = 64
OUT = 64

_B = 4096
_L = 200
_LT = _L // 8
_BT = _B // 128


_BN = 16384


def _transform_body(tT_ref, w_ref, b_ref, o_ref):
    h = jnp.maximum(tT_ref[...], 0.0)
    res = lax.dot_general(h, w_ref[...], (((0,), (1,)), ((), ())),
                          preferred_element_type=jnp.float32)
    o_ref[:, 0:OUT] = res + b_ref[...]


def _transform_table(tableT, W, b):
    n_rows = tableT.shape[1]
    grid = (n_rows + _BN - 1) // _BN
    return pl.pallas_call(
        _transform_body,
        grid=(grid,),
        in_specs=[
            pl.BlockSpec((HIDDEN, _BN), lambda i: (0, i)),
            pl.BlockSpec((OUT, HIDDEN), lambda i: (0, 0)),
            pl.BlockSpec((1, OUT), lambda i: (0, 0)),
        ],
        out_specs=pl.BlockSpec((_BN, 2 * OUT), lambda i: (i, 0)),
        out_shape=jax.ShapeDtypeStruct((n_rows, 2 * OUT), jnp.float32),
    )(tableT, W, b.reshape(1, OUT))


def _make_gather():
    mesh = plsc.VectorSubcoreMesh(core_axis_name="c", subcore_axis_name="s")

    @functools.partial(
        pl.kernel,
        mesh=mesh,
        out_type=jax.ShapeDtypeStruct((_L * _B // 8, 8, 128), jnp.float32),
        scratch_types=[
            pltpu.VMEM((_LT, 8, 128), jnp.int32),
            pltpu.VMEM((128, OUT), jnp.float32),
            pltpu.VMEM((128, OUT), jnp.float32),
            pltpu.VMEM((16, 8, OUT), jnp.float32),
            pltpu.VMEM((16, 8, OUT), jnp.float32),
            pltpu.SemaphoreType.DMA,
            pltpu.SemaphoreType.DMA,
            pltpu.SemaphoreType.DMA,
            pltpu.SemaphoreType.DMA,
        ],
        compiler_params=pltpu.CompilerParams(use_tc_tiling_on_sc=False),
    )
    def gather(table2_hbm, idx_hbm, out_hbm, idx_v, rows_a, rows_b, cb_a, cb_b,
               g0, g1, w0, w1):
        wid = lax.axis_index("s") * 2 + lax.axis_index("c")

        for lt in range(_LT):
            pltpu.sync_copy(idx_hbm.at[lt, wid], idx_v.at[lt])

        def dbl(lt, c):
            for lr in range(8):
                for v in range(8):
                    sl = pl.ds(16 * v, 16)
                    idx_v[lt, lr, sl] = idx_v[lt, lr, sl] * 2
            return c

        lax.fori_loop(0, _LT, dbl, 0, unroll=False)

        gsem = (g0, g1)
        wsem = (w0, w1)
        rbuf = (rows_a, rows_b)
        cbuf = (cb_a, cb_b)
        base = wid * 128

        def fire_gather(l, buf):
            pltpu.async_copy(table2_hbm.at[idx_v.at[l // 8, l % 8]],
                             rbuf[buf], gsem[buf])

        def wait_gather(l, buf):
            pltpu.make_async_copy(table2_hbm.at[idx_v.at[l // 8, l % 8]],
                                  rbuf[buf], gsem[buf]).wait()

        def fire_write(l, buf):
            pltpu.async_copy(cbuf[buf],
                             out_hbm.at[pl.ds((l * _B + base) // 8, 16), :,
                                        pl.ds(0, OUT)],
                             wsem[buf])

        def wait_write(l, buf):
            pltpu.make_async_copy(cbuf[buf],
                                  out_hbm.at[pl.ds((l * _B + base) // 8, 16), :,
                                             pl.ds(0, OUT)],
                                  wsem[buf]).wait()

        def compact(buf):
            rows = rbuf[buf]
            cb = cbuf[buf]
            for r in range(128):
                for q in range(4):
                    cb[r // 8, r % 8, pl.ds(q * 16, 16)] = rows[r, pl.ds(q * 16, 16)]

        def step(l, buf, last):
            wait_gather(l, buf)
            if not last:
                wait_write(l - 1, 1 - buf)
                fire_gather(l + 1, 1 - buf)
            compact(buf)
            fire_write(l, buf)

        fire_gather(jnp.int32(0), 0)
        wait_gather(jnp.int32(0), 0)
        fire_gather(jnp.int32(1), 1)
        compact(0)
        fire_write(jnp.int32(0), 0)

        def body(i, c):
            step(2 * i + 1, 1, False)
            step(2 * i + 2, 0, False)
            return c

        lax.fori_loop(0, (_L - 2) // 2, body, 0, unroll=False)
        step(jnp.int32(_L - 1), 1, True)
        wait_write(jnp.int32(_L - 2), 0)
        wait_write(jnp.int32(_L - 1), 1)

    return gather


_LBLK = 8


def _emit_body(e_ref, i_ref, o_ref):
    for k in range(_LBLK):
        a = e_ref[k].reshape(_B, 128)[:, 0:OUT]
        o_ref[k] = lax.dot_general(i_ref[...], a,
                                   (((1,), (1,)), ((), ())),
                                   preferred_element_type=jnp.float32)


def _emit(embL, ident):
    emb4 = embL.reshape(_L, _B // 8, 8, 128)
    return pl.pallas_call(
        _emit_body,
        grid=(_L // _LBLK,),
        in_specs=[
            pl.BlockSpec((_LBLK, _B // 8, 8, 128), lambda l: (l, 0, 0, 0)),
            pl.BlockSpec((OUT, OUT), lambda l: (0, 0)),
        ],
        out_specs=pl.BlockSpec((_LBLK, OUT, _B), lambda l: (l, 0, 0)),
        out_shape=jax.ShapeDtypeStruct((_L, OUT, _B), jnp.float32),
    )(emb4, ident)


def kernel(inputs, table, W, b):
    tableT = table.T
    table2 = _transform_table(tableT, W, b)

    idx5 = (inputs.astype(jnp.int32).T
            .reshape(_LT, 8, _BT, 128)
            .transpose(0, 2, 1, 3))

    table2h = table2.reshape(2 * D_MODEL, OUT)
    embL = _make_gather()(table2h, idx5)

    ident = jnp.eye(OUT, dtype=jnp.float32)
    out3 = _emit(embL, ident)

    return out3.transpose(2, 0, 1)
